# Initial kernel scaffold; baseline (speedup 1.0000x reference)
import jax
import jax.numpy as jnp
from jax.experimental import pallas as pl


def kernel(x):
    raise NotImplementedError("write your pallas kernel here")


# speedup vs baseline: 1.0006x; 1.0006x over previous
import jax
import jax.numpy as jnp
from jax.experimental import pallas as pl

_A = -1.0
_B = 1.0


def _block_kernel(x_ref, o_ref):
    x = x_ref[...]
    cond = (x >= _A) & (x <= _B)
    o_ref[...] = jnp.where(cond, jnp.sin(x), 0.0)


def kernel(x):
    orig_shape = x.shape
    x2 = x.reshape(-1, orig_shape[-1])
    m, n = x2.shape
    bm = 512
    out = pl.pallas_call(
        _block_kernel,
        grid=(m // bm,),
        in_specs=[pl.BlockSpec((bm, n), lambda i: (i, 0))],
        out_specs=pl.BlockSpec((bm, n), lambda i: (i, 0)),
        out_shape=jax.ShapeDtypeStruct((m, n), x.dtype),
    )(x2)
    return out.reshape(orig_shape)


# speedup vs baseline: 5.1032x; 5.1004x over previous
import jax
import jax.numpy as jnp
from jax.experimental import pallas as pl

_A = -1.0
_B = 1.0


_C3 = -1.0 / 6.0
_C5 = 1.0 / 120.0
_C7 = -1.0 / 5040.0
_C9 = 1.0 / 362880.0


def _block_kernel(x_ref, o_ref):
    x = x_ref[...]
    cond = (x >= _A) & (x <= _B)
    x2 = x * x
    p = _C3 + x2 * (_C5 + x2 * (_C7 + x2 * _C9))
    s = x + (x * x2) * p
    o_ref[...] = jnp.where(cond, s, 0.0)


def kernel(x):
    orig_shape = x.shape
    x2 = x.reshape(-1, orig_shape[-1])
    m, n = x2.shape
    bm = 512
    out = pl.pallas_call(
        _block_kernel,
        grid=(m // bm,),
        in_specs=[pl.BlockSpec((bm, n), lambda i: (i, 0))],
        out_specs=pl.BlockSpec((bm, n), lambda i: (i, 0)),
        out_shape=jax.ShapeDtypeStruct((m, n), x.dtype),
    )(x2)
    return out.reshape(orig_shape)
